# baseline (speedup 1.0000x reference)
import jax
import jax.numpy as jnp
from jax import lax
from jax.experimental import pallas as pl
from jax.experimental.pallas import tpu as pltpu

_K = 1024
_D = 32
_N = 4096
_BT = 512
_SPLIT = 1024 // _BT
_GRID = _N // _BT


def _vq_body(xt_ref, w_ref, wsq_ref, idx_ref, loss_ref, qt_ref, acc_ref):
    xt = xt_ref[...]
    w2 = w_ref[...] * (-2.0)
    wsq = wsq_ref[...]
    s = xt * xt
    xsq = s[0:1, :]
    for c in range(1, _D):
        xsq = xsq + s[c:c + 1, :]
    mm2 = lax.dot_general(w2, xt, (((1,), (0,)), ((), ())),
                          preferred_element_type=jnp.float32)
    d = (wsq + xsq) + mm2
    m = jnp.min(d, axis=0, keepdims=True)
    iota = lax.broadcasted_iota(jnp.int32, (_K, 1), 0).astype(jnp.float32)
    idxf = jnp.min(jnp.where(d == m, iota, jnp.float32(_K)), axis=0,
                   keepdims=True)
    idx_ref[...] = idxf.astype(jnp.int32).reshape(1, 1, _BT)
    onehot = jnp.where(iota == idxf, 1.0, 0.0)
    qt = lax.dot_general(w2, onehot, (((0,), (0,)), ((), ())),
                         preferred_element_type=jnp.float32) * (-0.5)
    qt_ref[...] = qt.reshape(1, _D, _BT)
    part = jnp.sum(m)

    i = pl.program_id(0)

    @pl.when(i == 0)
    def _init():
        acc_ref[0] = 0.0

    acc_ref[0] += part

    @pl.when(i == _GRID - 1)
    def _fini():
        loss_ref[0, 0] = acc_ref[0] * (1.25 / (_N * _D))


def _vq(x_cn, weight, wsq):
    return pl.pallas_call(
        _vq_body,
        grid=(_GRID,),
        in_specs=[
            pl.BlockSpec((_D, _BT), lambda i: (i // _SPLIT, i % _SPLIT)),
            pl.BlockSpec((_K, _D), lambda i: (0, 0)),
            pl.BlockSpec((_K, 1), lambda i: (0, 0)),
        ],
        out_specs=[
            pl.BlockSpec((1, 1, _BT), lambda i: (i // _SPLIT, 0, i % _SPLIT)),
            pl.BlockSpec(memory_space=pltpu.SMEM),
            pl.BlockSpec((1, _D, _BT), lambda i: (i // _SPLIT, 0, i % _SPLIT)),
        ],
        out_shape=[
            jax.ShapeDtypeStruct((4, 1, 1024), jnp.int32),
            jax.ShapeDtypeStruct((1, 1), jnp.float32),
            jax.ShapeDtypeStruct((4, _D, 1024), jnp.float32),
        ],
        scratch_shapes=[pltpu.SMEM((1,), jnp.float32)],
    )(x_cn, weight, wsq)


def kernel(inputs, weight):
    x_cn = inputs.reshape(4 * _D, 32 * 32)
    wsq = jnp.sum(weight ** 2, axis=1).reshape(_K, 1)
    idx3, loss, qt = _vq(x_cn, weight, wsq)
    quantized_st = qt.reshape(4, _D, 32, 32)
    return quantized_st, loss[0, 0], idx3.reshape(4, 32, 32)
